# baseline (speedup 1.0000x reference)
import jax
import jax.numpy as jnp
from jax.experimental import pallas as pl
from jax.experimental.pallas import tpu as pltpu

TOP_P = 0.5
_LOG_EPS = 1e-10
_CV_EPS = 1e-10


def _main_body(x_ref, w_ref, m_ref, out_ref, v_ref, ent_ref):
    lb = pl.program_id(0)
    bh = pl.program_id(1)
    xb = x_ref[0]
    w = w_ref[...]
    BL = xb.shape[0]
    E = w.shape[0]

    logits = jax.lax.dot_general(
        xb, w, (((1,), (1,)), ((), ())), preferred_element_type=jnp.float32)
    mx = jnp.max(logits, axis=1, keepdims=True)
    ex = jnp.exp(logits - mx)
    p = ex / jnp.sum(ex, axis=1, keepdims=True)

    ent = -jnp.sum(p * jnp.log(p + _LOG_EPS))

    col = jax.lax.broadcasted_iota(jnp.int32, p.shape, 1)
    s_before_cols = []
    rank_cols = []
    for e in range(E):
        pe = p[:, e:e + 1]
        ahead = (p > pe) | ((p == pe) & (col < e))
        s_before_cols.append(
            jnp.sum(jnp.where(ahead, p, 0.0), axis=1, keepdims=True))
        rank_cols.append(
            jnp.sum(ahead.astype(jnp.float32), axis=1, keepdims=True))
    s_before = jnp.concatenate(s_before_cols, axis=1)
    rank = jnp.concatenate(rank_cols, axis=1)
    gates = (s_before <= TOP_P).astype(jnp.float32)

    pg = p * gates
    vc = jnp.concatenate(
        [jnp.sum(jnp.where(rank == r, pg, 0.0), axis=1, keepdims=True)
         for r in range(E)], axis=1)

    @pl.when(bh == 0)
    def _():
        v_ref[...] = vc

    @pl.when(bh > 0)
    def _():
        v_ref[...] = v_ref[...] + vc

    first = jnp.logical_and(lb == 0, bh == 0)

    @pl.when(first)
    def _():
        ent_ref[0, 0] = ent

    @pl.when(jnp.logical_not(first))
    def _():
        ent_ref[0, 0] = ent_ref[0, 0] + ent

    mb = m_ref[...]
    acc = gates[:, 0:1] * mb[:, 0, :]
    for i in range(1, E):
        acc = acc + gates[:, i:i + 1] * mb[:, i, :]
    rows = lb * BL + jax.lax.broadcasted_iota(jnp.int32, acc.shape, 0)
    cols = jax.lax.broadcasted_iota(jnp.int32, acc.shape, 1)
    acc = jnp.where(rows == cols, acc + 1.0, acc)
    out_ref[0] = acc


def _loss_body(v_ref, ent_ref, loss_ref, *, n_rows_experts):
    v = v_ref[...]
    n = v.shape[0] * v.shape[1]
    mean = jnp.sum(v) / n
    var = jnp.sum((v - mean) ** 2) / (n - 1)
    loss_imp = var / (mean * mean + _CV_EPS)
    loss_dyn = ent_ref[0, 0] / n_rows_experts
    loss_ref[0, 0] = loss_imp + 0.1 * loss_dyn


def kernel(x, masks, W_gate, W_noise):
    B, H, L, _ = x.shape
    E = W_gate.shape[0]
    BH = B * H
    BL = 256
    LB = L // BL
    xr = x.reshape(BH, L, L)

    out_bh, v, ent = pl.pallas_call(
        _main_body,
        grid=(LB, BH),
        in_specs=[
            pl.BlockSpec((1, BL, L), lambda lb, bh: (bh, lb, 0)),
            pl.BlockSpec((E, L), lambda lb, bh: (0, 0)),
            pl.BlockSpec((BL, E, L), lambda lb, bh: (lb, 0, 0)),
        ],
        out_specs=[
            pl.BlockSpec((1, BL, L), lambda lb, bh: (bh, lb, 0)),
            pl.BlockSpec((BL, E), lambda lb, bh: (lb, 0)),
            pl.BlockSpec((1, 1), lambda lb, bh: (0, 0),
                         memory_space=pltpu.SMEM),
        ],
        out_shape=[
            jax.ShapeDtypeStruct((BH, L, L), jnp.float32),
            jax.ShapeDtypeStruct((L, E), jnp.float32),
            jax.ShapeDtypeStruct((1, 1), jnp.float32),
        ],
    )(xr, W_gate, masks)

    import functools
    loss2 = pl.pallas_call(
        functools.partial(_loss_body, n_rows_experts=BH * E),
        in_specs=[
            pl.BlockSpec((L, E), lambda: (0, 0)),
            pl.BlockSpec((1, 1), lambda: (0, 0), memory_space=pltpu.SMEM),
        ],
        out_specs=pl.BlockSpec((1, 1), lambda: (0, 0),
                               memory_space=pltpu.SMEM),
        out_shape=jax.ShapeDtypeStruct((1, 1), jnp.float32),
    )(v, ent)

    return out_bh.reshape(B, H, L, L), loss2[0, 0]


# speedup vs baseline: 2.1979x; 2.1979x over previous
import jax
import jax.numpy as jnp
from jax.experimental import pallas as pl
from jax.experimental.pallas import tpu as pltpu

TOP_P = 0.5
_LOG_EPS = 1e-10
_CV_EPS = 1e-10


def _main_body(x_ref, w_ref, m_ref, out_ref, v_ref, ent_ref):
    lb = pl.program_id(0)
    bh = pl.program_id(1)
    xb = x_ref[0]
    w = w_ref[...]
    BL = xb.shape[0]
    E = w.shape[0]

    logits = jax.lax.dot_general(
        xb, w, (((1,), (1,)), ((), ())), preferred_element_type=jnp.float32)
    mx = jnp.max(logits, axis=1, keepdims=True)
    ex = jnp.exp(logits - mx)
    p = ex / jnp.sum(ex, axis=1, keepdims=True)

    ent = -jnp.sum(p * jnp.log(p + _LOG_EPS))

    col = jax.lax.broadcasted_iota(jnp.int32, p.shape, 1)
    s_before_cols = []
    rank_cols = []
    for e in range(E):
        pe = p[:, e:e + 1]
        ahead = (p > pe) | ((p == pe) & (col < e))
        s_before_cols.append(
            jnp.sum(jnp.where(ahead, p, 0.0), axis=1, keepdims=True))
        rank_cols.append(
            jnp.sum(ahead.astype(jnp.float32), axis=1, keepdims=True))
    s_before = jnp.concatenate(s_before_cols, axis=1)
    rank = jnp.concatenate(rank_cols, axis=1)
    gates = (s_before <= TOP_P).astype(jnp.float32)

    pg = p * gates
    vc = jnp.concatenate(
        [jnp.sum(jnp.where(rank == r, pg, 0.0), axis=1, keepdims=True)
         for r in range(E)], axis=1)

    @pl.when(bh == 0)
    def _():
        v_ref[...] = vc

    @pl.when(bh > 0)
    def _():
        v_ref[...] = v_ref[...] + vc

    first = jnp.logical_and(lb == 0, bh == 0)

    @pl.when(first)
    def _():
        ent_ref[0, 0] = ent

    @pl.when(jnp.logical_not(first))
    def _():
        ent_ref[0, 0] = ent_ref[0, 0] + ent

    mb = m_ref[...]
    Lout = out_ref.shape[2]
    acc = gates[:, 0:1] * mb[:, 0:Lout]
    for i in range(1, E):
        acc = acc + gates[:, i:i + 1] * mb[:, i * Lout:(i + 1) * Lout]
    rows = lb * BL + jax.lax.broadcasted_iota(jnp.int32, acc.shape, 0)
    cols = jax.lax.broadcasted_iota(jnp.int32, acc.shape, 1)
    acc = jnp.where(rows == cols, acc + 1.0, acc)
    out_ref[0] = acc


def _loss_body(v_ref, ent_ref, loss_ref, *, n_rows_experts):
    v = v_ref[...]
    n = v.shape[0] * v.shape[1]
    mean = jnp.sum(v) / n
    var = jnp.sum((v - mean) ** 2) / (n - 1)
    loss_imp = var / (mean * mean + _CV_EPS)
    loss_dyn = ent_ref[0, 0] / n_rows_experts
    loss_ref[0, 0] = loss_imp + 0.1 * loss_dyn


def kernel(x, masks, W_gate, W_noise):
    B, H, L, _ = x.shape
    E = W_gate.shape[0]
    BH = B * H
    BL = 256
    LB = L // BL
    xr = x.reshape(BH, L, L)

    out_bh, v, ent = pl.pallas_call(
        _main_body,
        grid=(LB, BH),
        in_specs=[
            pl.BlockSpec((1, BL, L), lambda lb, bh: (bh, lb, 0)),
            pl.BlockSpec((E, L), lambda lb, bh: (0, 0)),
            pl.BlockSpec((BL, E * L), lambda lb, bh: (lb, 0)),
        ],
        out_specs=[
            pl.BlockSpec((1, BL, L), lambda lb, bh: (bh, lb, 0)),
            pl.BlockSpec((BL, E), lambda lb, bh: (lb, 0)),
            pl.BlockSpec((1, 1), lambda lb, bh: (0, 0),
                         memory_space=pltpu.SMEM),
        ],
        out_shape=[
            jax.ShapeDtypeStruct((BH, L, L), jnp.float32),
            jax.ShapeDtypeStruct((L, E), jnp.float32),
            jax.ShapeDtypeStruct((1, 1), jnp.float32),
        ],
    )(xr, W_gate, masks.reshape(L, E * L))

    import functools
    loss2 = pl.pallas_call(
        functools.partial(_loss_body, n_rows_experts=BH * E),
        in_specs=[
            pl.BlockSpec((L, E), lambda: (0, 0)),
            pl.BlockSpec((1, 1), lambda: (0, 0), memory_space=pltpu.SMEM),
        ],
        out_specs=pl.BlockSpec((1, 1), lambda: (0, 0),
                               memory_space=pltpu.SMEM),
        out_shape=jax.ShapeDtypeStruct((1, 1), jnp.float32),
    )(v, ent)

    return out_bh.reshape(B, H, L, L), loss2[0, 0]


# speedup vs baseline: 3.6802x; 1.6744x over previous
import functools

import jax
import jax.numpy as jnp
from jax.experimental import pallas as pl
from jax.experimental.pallas import tpu as pltpu

TOP_P = 0.5
_LOG_EPS = 1e-10
_CV_EPS = 1e-10


def _main_body(x_ref, w_ref, m_ref, out_ref, v_ref, ent_ref):
    lb = pl.program_id(0)
    bh = pl.program_id(1)
    xb = x_ref[0]
    w = w_ref[...]
    BL = xb.shape[0]
    E = w.shape[0]

    lt = jax.lax.dot_general(
        w, xb, (((1,), (1,)), ((), ())), preferred_element_type=jnp.float32)
    mx = jnp.max(lt, axis=0, keepdims=True)
    ex = jnp.exp(lt - mx)
    p = ex / jnp.sum(ex, axis=0, keepdims=True)

    ent = -jnp.sum(p * jnp.log(p + _LOG_EPS))

    s_before_rows = []
    rank_rows = []
    for e in range(E):
        pe = p[e:e + 1, :]
        if e == 0:
            ahead = (p > pe)
        else:
            gt = p > pe
            eq_lt = (p == pe) & (jax.lax.broadcasted_iota(
                jnp.int32, p.shape, 0) < e)
            ahead = gt | eq_lt
        s_before_rows.append(
            jnp.sum(jnp.where(ahead, p, 0.0), axis=0, keepdims=True))
        rank_rows.append(
            jnp.sum(ahead.astype(jnp.float32), axis=0, keepdims=True))
    s_before = jnp.concatenate(s_before_rows, axis=0)
    rank = jnp.concatenate(rank_rows, axis=0)
    gates_t = (s_before <= TOP_P).astype(jnp.float32)

    pg = p * gates_t
    vc = jnp.concatenate(
        [jnp.sum(jnp.where(rank == r, pg, 0.0), axis=0, keepdims=True)
         for r in range(E)], axis=0)

    @pl.when(bh == 0)
    def _():
        v_ref[...] = vc

    @pl.when(bh > 0)
    def _():
        v_ref[...] = v_ref[...] + vc

    first = jnp.logical_and(lb == 0, bh == 0)

    @pl.when(first)
    def _():
        ent_ref[0, 0] = ent

    @pl.when(jnp.logical_not(first))
    def _():
        ent_ref[0, 0] = ent_ref[0, 0] + ent

    gates = gates_t.T
    mb = m_ref[...]
    Lout = out_ref.shape[2]
    acc = gates[:, 0:1] * mb[:, 0:Lout]
    for i in range(1, E):
        acc = acc + gates[:, i:i + 1] * mb[:, i * Lout:(i + 1) * Lout]
    rows = lb * BL + jax.lax.broadcasted_iota(jnp.int32, acc.shape, 0)
    cols = jax.lax.broadcasted_iota(jnp.int32, acc.shape, 1)
    acc = jnp.where(rows == cols, acc + 1.0, acc)
    out_ref[0] = acc


def _loss_body(v_ref, ent_ref, loss_ref, *, n_rows_experts):
    v = v_ref[...]
    n = v.shape[0] * v.shape[1]
    mean = jnp.sum(v) / n
    var = jnp.sum((v - mean) ** 2) / (n - 1)
    loss_imp = var / (mean * mean + _CV_EPS)
    loss_dyn = ent_ref[0, 0] / n_rows_experts
    loss_ref[0, 0] = loss_imp + 0.1 * loss_dyn


def kernel(x, masks, W_gate, W_noise):
    B, H, L, _ = x.shape
    E = W_gate.shape[0]
    BH = B * H
    BL = 256
    LB = L // BL
    xr = x.reshape(BH, L, L)

    out_bh, v, ent = pl.pallas_call(
        _main_body,
        grid=(LB, BH),
        in_specs=[
            pl.BlockSpec((1, BL, L), lambda lb, bh: (bh, lb, 0)),
            pl.BlockSpec((E, L), lambda lb, bh: (0, 0)),
            pl.BlockSpec((BL, E * L), lambda lb, bh: (lb, 0)),
        ],
        out_specs=[
            pl.BlockSpec((1, BL, L), lambda lb, bh: (bh, lb, 0)),
            pl.BlockSpec((E, BL), lambda lb, bh: (0, lb)),
            pl.BlockSpec((1, 1), lambda lb, bh: (0, 0),
                         memory_space=pltpu.SMEM),
        ],
        out_shape=[
            jax.ShapeDtypeStruct((BH, L, L), jnp.float32),
            jax.ShapeDtypeStruct((E, L), jnp.float32),
            jax.ShapeDtypeStruct((1, 1), jnp.float32),
        ],
    )(xr, W_gate, masks.reshape(L, E * L))

    loss2 = pl.pallas_call(
        functools.partial(_loss_body, n_rows_experts=BH * E),
        in_specs=[
            pl.BlockSpec((E, L), lambda: (0, 0)),
            pl.BlockSpec((1, 1), lambda: (0, 0), memory_space=pltpu.SMEM),
        ],
        out_specs=pl.BlockSpec((1, 1), lambda: (0, 0),
                               memory_space=pltpu.SMEM),
        out_shape=jax.ShapeDtypeStruct((1, 1), jnp.float32),
    )(v, ent)

    return out_bh.reshape(B, H, L, L), loss2[0, 0]


# speedup vs baseline: 4.1212x; 1.1198x over previous
import functools

import jax
import jax.numpy as jnp
from jax.experimental import pallas as pl
from jax.experimental.pallas import tpu as pltpu

TOP_P = 0.5
_LOG_EPS = 1e-10
_CV_EPS = 1e-10


def _main_body(x_ref, w_ref, m_ref, out_ref, v_ref, ent_ref):
    lb = pl.program_id(0)
    bh = pl.program_id(1)
    xb = x_ref[0]
    w = w_ref[...]
    BL = xb.shape[0]
    E = w.shape[0]

    lt = jax.lax.dot_general(
        w, xb, (((1,), (1,)), ((), ())), preferred_element_type=jnp.float32)
    mx = jnp.max(lt, axis=0, keepdims=True)
    ex = jnp.exp(lt - mx)
    p = ex / jnp.sum(ex, axis=0, keepdims=True)

    ent = -jnp.sum(p * jnp.log(p + _LOG_EPS))

    s_before_rows = []
    rank_rows = []
    for e in range(E):
        pe = p[e:e + 1, :]
        if e == 0:
            ahead = (p > pe)
        else:
            gt = p > pe
            eq_lt = (p == pe) & (jax.lax.broadcasted_iota(
                jnp.int32, p.shape, 0) < e)
            ahead = gt | eq_lt
        s_before_rows.append(
            jnp.sum(jnp.where(ahead, p, 0.0), axis=0, keepdims=True))
        rank_rows.append(
            jnp.sum(ahead.astype(jnp.float32), axis=0, keepdims=True))
    s_before = jnp.concatenate(s_before_rows, axis=0)
    rank = jnp.concatenate(rank_rows, axis=0)
    gates_t = (s_before <= TOP_P).astype(jnp.float32)

    pg = p * gates_t
    vc = jnp.concatenate(
        [jnp.sum(jnp.where(rank == r, pg, 0.0), axis=0, keepdims=True)
         for r in range(E)], axis=0)

    @pl.when(bh == 0)
    def _():
        v_ref[...] = vc

    @pl.when(bh > 0)
    def _():
        v_ref[...] = v_ref[...] + vc

    first = jnp.logical_and(lb == 0, bh == 0)

    @pl.when(first)
    def _():
        ent_ref[0, 0] = ent

    @pl.when(jnp.logical_not(first))
    def _():
        ent_ref[0, 0] = ent_ref[0, 0] + ent

    gates = gates_t.T
    mb = m_ref[...]
    Lout = out_ref.shape[2]
    acc = gates[:, 0:1] * mb[:, 0:Lout]
    for i in range(1, E):
        acc = acc + gates[:, i:i + 1] * mb[:, i * Lout:(i + 1) * Lout]
    rows = lb * BL + jax.lax.broadcasted_iota(jnp.int32, acc.shape, 0)
    cols = jax.lax.broadcasted_iota(jnp.int32, acc.shape, 1)
    acc = jnp.where(rows == cols, acc + 1.0, acc)
    out_ref[0] = acc


def _loss_body(v_ref, ent_ref, loss_ref, *, n_rows_experts):
    v = v_ref[...]
    n = v.shape[0] * v.shape[1]
    mean = jnp.sum(v) / n
    var = jnp.sum((v - mean) ** 2) / (n - 1)
    loss_imp = var / (mean * mean + _CV_EPS)
    loss_dyn = ent_ref[0, 0] / n_rows_experts
    loss_ref[0, 0] = loss_imp + 0.1 * loss_dyn


def kernel(x, masks, W_gate, W_noise):
    B, H, L, _ = x.shape
    E = W_gate.shape[0]
    BH = B * H
    BL = 512
    LB = L // BL
    xr = x.reshape(BH, L, L)

    out_bh, v, ent = pl.pallas_call(
        _main_body,
        grid=(LB, BH),
        in_specs=[
            pl.BlockSpec((1, BL, L), lambda lb, bh: (bh, lb, 0)),
            pl.BlockSpec((E, L), lambda lb, bh: (0, 0)),
            pl.BlockSpec((BL, E * L), lambda lb, bh: (lb, 0)),
        ],
        out_specs=[
            pl.BlockSpec((1, BL, L), lambda lb, bh: (bh, lb, 0)),
            pl.BlockSpec((E, BL), lambda lb, bh: (0, lb)),
            pl.BlockSpec((1, 1), lambda lb, bh: (0, 0),
                         memory_space=pltpu.SMEM),
        ],
        out_shape=[
            jax.ShapeDtypeStruct((BH, L, L), jnp.float32),
            jax.ShapeDtypeStruct((E, L), jnp.float32),
            jax.ShapeDtypeStruct((1, 1), jnp.float32),
        ],
    )(xr, W_gate, masks.reshape(L, E * L))

    loss2 = pl.pallas_call(
        functools.partial(_loss_body, n_rows_experts=BH * E),
        in_specs=[
            pl.BlockSpec((E, L), lambda: (0, 0)),
            pl.BlockSpec((1, 1), lambda: (0, 0), memory_space=pltpu.SMEM),
        ],
        out_specs=pl.BlockSpec((1, 1), lambda: (0, 0),
                               memory_space=pltpu.SMEM),
        out_shape=jax.ShapeDtypeStruct((1, 1), jnp.float32),
    )(v, ent)

    return out_bh.reshape(B, H, L, L), loss2[0, 0]


# speedup vs baseline: 4.1218x; 1.0001x over previous
import functools

import jax
import jax.numpy as jnp
from jax.experimental import pallas as pl
from jax.experimental.pallas import tpu as pltpu

TOP_P = 0.5
_LOG_EPS = 1e-10
_CV_EPS = 1e-10


def _main_body(x_ref, w_ref, m_ref, out_ref, v_ref, ent_ref):
    lb = pl.program_id(0)
    bh = pl.program_id(1)
    xb = x_ref[0]
    w = w_ref[...]
    BL = xb.shape[0]
    E = w.shape[0]

    lt = jax.lax.dot_general(
        w, xb, (((1,), (1,)), ((), ())), preferred_element_type=jnp.float32)
    mx = jnp.max(lt, axis=0, keepdims=True)
    ex = jnp.exp(lt - mx)
    p = ex / jnp.sum(ex, axis=0, keepdims=True)

    ent = -jnp.sum(p * jnp.log(p + _LOG_EPS))

    s_before_rows = []
    rank_rows = []
    for e in range(E):
        pe = p[e:e + 1, :]
        if e == 0:
            ahead = (p > pe)
        else:
            gt = p > pe
            eq_lt = (p == pe) & (jax.lax.broadcasted_iota(
                jnp.int32, p.shape, 0) < e)
            ahead = gt | eq_lt
        s_before_rows.append(
            jnp.sum(jnp.where(ahead, p, 0.0), axis=0, keepdims=True))
        rank_rows.append(
            jnp.sum(ahead.astype(jnp.float32), axis=0, keepdims=True))
    s_before = jnp.concatenate(s_before_rows, axis=0)
    rank = jnp.concatenate(rank_rows, axis=0)
    gates_t = (s_before <= TOP_P).astype(jnp.float32)

    pg = p * gates_t
    vc = jnp.concatenate(
        [jnp.sum(jnp.where(rank == r, pg, 0.0), axis=0, keepdims=True)
         for r in range(E)], axis=0)

    @pl.when(bh == 0)
    def _():
        v_ref[...] = vc

    @pl.when(bh > 0)
    def _():
        v_ref[...] = v_ref[...] + vc

    @pl.when(bh == 0)
    def _():
        ent_ref[0, 0, 0] = ent

    @pl.when(bh > 0)
    def _():
        ent_ref[0, 0, 0] = ent_ref[0, 0, 0] + ent

    gates = gates_t.T
    mb = m_ref[...]
    Lout = out_ref.shape[2]
    acc = gates[:, 0:1] * mb[:, 0:Lout]
    for i in range(1, E):
        acc = acc + gates[:, i:i + 1] * mb[:, i * Lout:(i + 1) * Lout]
    rows = lb * BL + jax.lax.broadcasted_iota(jnp.int32, acc.shape, 0)
    cols = jax.lax.broadcasted_iota(jnp.int32, acc.shape, 1)
    acc = jnp.where(rows == cols, acc + 1.0, acc)
    out_ref[0] = acc


def _loss_body(v_ref, ent_ref, loss_ref, *, n_rows_experts):
    v = v_ref[...]
    n = v.shape[0] * v.shape[1]
    mean = jnp.sum(v) / n
    var = jnp.sum((v - mean) ** 2) / (n - 1)
    loss_imp = var / (mean * mean + _CV_EPS)
    ent = ent_ref[0, 0, 0]
    for i in range(1, ent_ref.shape[0]):
        ent = ent + ent_ref[i, 0, 0]
    loss_dyn = ent / n_rows_experts
    loss_ref[0, 0] = loss_imp + 0.1 * loss_dyn


def kernel(x, masks, W_gate, W_noise):
    B, H, L, _ = x.shape
    E = W_gate.shape[0]
    BH = B * H
    BL = 512
    LB = L // BL
    xr = x.reshape(BH, L, L)

    out_bh, v, ent = pl.pallas_call(
        _main_body,
        grid=(LB, BH),
        in_specs=[
            pl.BlockSpec((1, BL, L), lambda lb, bh: (bh, lb, 0)),
            pl.BlockSpec((E, L), lambda lb, bh: (0, 0)),
            pl.BlockSpec((BL, E * L), lambda lb, bh: (lb, 0)),
        ],
        out_specs=[
            pl.BlockSpec((1, BL, L), lambda lb, bh: (bh, lb, 0)),
            pl.BlockSpec((E, BL), lambda lb, bh: (0, lb)),
            pl.BlockSpec((1, 1, 1), lambda lb, bh: (lb, 0, 0),
                         memory_space=pltpu.SMEM),
        ],
        out_shape=[
            jax.ShapeDtypeStruct((BH, L, L), jnp.float32),
            jax.ShapeDtypeStruct((E, L), jnp.float32),
            jax.ShapeDtypeStruct((LB, 1, 1), jnp.float32),
        ],
        compiler_params=pltpu.CompilerParams(
            dimension_semantics=("parallel", "arbitrary")),
    )(xr, W_gate, masks.reshape(L, E * L))

    loss2 = pl.pallas_call(
        functools.partial(_loss_body, n_rows_experts=BH * E),
        in_specs=[
            pl.BlockSpec((E, L), lambda: (0, 0)),
            pl.BlockSpec((LB, 1, 1), lambda: (0, 0, 0),
                         memory_space=pltpu.SMEM),
        ],
        out_specs=pl.BlockSpec((1, 1), lambda: (0, 0),
                               memory_space=pltpu.SMEM),
        out_shape=jax.ShapeDtypeStruct((1, 1), jnp.float32),
    )(v, ent)

    return out_bh.reshape(B, H, L, L), loss2[0, 0]


# speedup vs baseline: 4.2148x; 1.0226x over previous
import functools

import jax
import jax.numpy as jnp
from jax.experimental import pallas as pl
from jax.experimental.pallas import tpu as pltpu

TOP_P = 0.5
_LOG_EPS = 1e-10
_CV_EPS = 1e-10


def _main_body(x_ref, w_ref, m_ref, out_ref, v_ref, ent_ref, scr_ref):
    lb = pl.program_id(0)
    bh = pl.program_id(1)
    xb = x_ref[0]
    w = w_ref[...]
    BL = xb.shape[0]
    E = w.shape[0]

    lt = jax.lax.dot_general(
        w, xb, (((1,), (1,)), ((), ())), preferred_element_type=jnp.float32)
    mx = jnp.max(lt, axis=0, keepdims=True)
    ex = jnp.exp(lt - mx)
    p = ex / jnp.sum(ex, axis=0, keepdims=True)

    ent = -jnp.sum(p * jnp.log(p + _LOG_EPS))

    s_before_rows = []
    rank_rows = []
    for e in range(E):
        pe = p[e:e + 1, :]
        if e == 0:
            ahead = (p > pe)
        else:
            gt = p > pe
            eq_lt = (p == pe) & (jax.lax.broadcasted_iota(
                jnp.int32, p.shape, 0) < e)
            ahead = gt | eq_lt
        s_before_rows.append(
            jnp.sum(jnp.where(ahead, p, 0.0), axis=0, keepdims=True))
        rank_rows.append(
            jnp.sum(ahead.astype(jnp.float32), axis=0, keepdims=True))
    s_before = jnp.concatenate(s_before_rows, axis=0)
    rank = jnp.concatenate(rank_rows, axis=0)
    gates_t = (s_before <= TOP_P).astype(jnp.float32)

    pg = p * gates_t
    vc = jnp.concatenate(
        [jnp.sum(jnp.where(rank == r, pg, 0.0), axis=0, keepdims=True)
         for r in range(E)], axis=0)

    @pl.when(bh == 0)
    def _():
        v_ref[...] = vc

    @pl.when(bh > 0)
    def _():
        v_ref[...] = v_ref[...] + vc

    @pl.when(bh == 0)
    def _():
        ent_ref[0, 0, 0] = ent

    @pl.when(bh > 0)
    def _():
        ent_ref[0, 0, 0] = ent_ref[0, 0, 0] + ent

    Lout = out_ref.shape[2]

    @pl.when(bh == 0)
    def _():
        for i in range(E):
            scr_ref[:, i * Lout:(i + 1) * Lout] = m_ref[:, i, :]

    gates = gates_t.T
    acc = gates[:, 0:1] * scr_ref[:, 0:Lout]
    for i in range(1, E):
        acc = acc + gates[:, i:i + 1] * scr_ref[:, i * Lout:(i + 1) * Lout]
    rows = lb * BL + jax.lax.broadcasted_iota(jnp.int32, acc.shape, 0)
    cols = jax.lax.broadcasted_iota(jnp.int32, acc.shape, 1)
    acc = jnp.where(rows == cols, acc + 1.0, acc)
    out_ref[0] = acc


def _loss_body(v_ref, ent_ref, loss_ref, *, n_rows_experts):
    v = v_ref[...]
    n = v.shape[0] * v.shape[1]
    mean = jnp.sum(v) / n
    var = jnp.sum((v - mean) ** 2) / (n - 1)
    loss_imp = var / (mean * mean + _CV_EPS)
    ent = ent_ref[0, 0, 0]
    for i in range(1, ent_ref.shape[0]):
        ent = ent + ent_ref[i, 0, 0]
    loss_dyn = ent / n_rows_experts
    loss_ref[0, 0] = loss_imp + 0.1 * loss_dyn


def kernel(x, masks, W_gate, W_noise):
    B, H, L, _ = x.shape
    E = W_gate.shape[0]
    BH = B * H
    BL = 256
    LB = L // BL
    xr = x.reshape(BH, L, L)

    out_bh, v, ent = pl.pallas_call(
        _main_body,
        grid=(LB, BH),
        in_specs=[
            pl.BlockSpec((1, BL, L), lambda lb, bh: (bh, lb, 0)),
            pl.BlockSpec((E, L), lambda lb, bh: (0, 0)),
            pl.BlockSpec((BL, E, L), lambda lb, bh: (lb, 0, 0)),
        ],
        out_specs=[
            pl.BlockSpec((1, BL, L), lambda lb, bh: (bh, lb, 0)),
            pl.BlockSpec((E, BL), lambda lb, bh: (0, lb)),
            pl.BlockSpec((1, 1, 1), lambda lb, bh: (lb, 0, 0),
                         memory_space=pltpu.SMEM),
        ],
        out_shape=[
            jax.ShapeDtypeStruct((BH, L, L), jnp.float32),
            jax.ShapeDtypeStruct((E, L), jnp.float32),
            jax.ShapeDtypeStruct((LB, 1, 1), jnp.float32),
        ],
        scratch_shapes=[pltpu.VMEM((BL, E * L), jnp.float32)],
        compiler_params=pltpu.CompilerParams(
            dimension_semantics=("parallel", "arbitrary")),
    )(xr, W_gate, masks)

    loss2 = pl.pallas_call(
        functools.partial(_loss_body, n_rows_experts=BH * E),
        in_specs=[
            pl.BlockSpec((E, L), lambda: (0, 0)),
            pl.BlockSpec((LB, 1, 1), lambda: (0, 0, 0),
                         memory_space=pltpu.SMEM),
        ],
        out_specs=pl.BlockSpec((1, 1), lambda: (0, 0),
                               memory_space=pltpu.SMEM),
        out_shape=jax.ShapeDtypeStruct((1, 1), jnp.float32),
    )(v, ent)

    return out_bh.reshape(B, H, L, L), loss2[0, 0]


# speedup vs baseline: 4.8649x; 1.1542x over previous
import functools

import jax
import jax.numpy as jnp
from jax.experimental import pallas as pl
from jax.experimental.pallas import tpu as pltpu

TOP_P = 0.5
_LOG_EPS = 1e-10
_CV_EPS = 1e-10


def _main_body(x_ref, w_ref, m_ref, out_ref, v_ref, ent_ref, scr_ref):
    lb = pl.program_id(0)
    bh = pl.program_id(1)
    xb = x_ref[0]
    w = w_ref[...]
    BL = xb.shape[0]
    E = w.shape[0]

    lt = jax.lax.dot_general(
        w, xb, (((1,), (1,)), ((), ())), preferred_element_type=jnp.float32)
    mx = jnp.max(lt, axis=0, keepdims=True)
    ex = jnp.exp(lt - mx)
    p = ex / jnp.sum(ex, axis=0, keepdims=True)

    ent = -jnp.sum(p * jnp.log(p + _LOG_EPS))

    s_before_rows = []
    rank_rows = []
    for e in range(E):
        pe = p[e:e + 1, :]
        if e == 0:
            ahead = (p > pe)
        else:
            gt = p > pe
            eq_lt = (p == pe) & (jax.lax.broadcasted_iota(
                jnp.int32, p.shape, 0) < e)
            ahead = gt | eq_lt
        s_before_rows.append(
            jnp.sum(jnp.where(ahead, p, 0.0), axis=0, keepdims=True))
        rank_rows.append(
            jnp.sum(ahead.astype(jnp.float32), axis=0, keepdims=True))
    s_before = jnp.concatenate(s_before_rows, axis=0)
    rank = jnp.concatenate(rank_rows, axis=0)
    gates_t = (s_before <= TOP_P).astype(jnp.float32)

    pg = p * gates_t
    vc = jnp.concatenate(
        [jnp.sum(jnp.where(rank == r, pg, 0.0), axis=0, keepdims=True)
         for r in range(E)], axis=0)

    @pl.when(bh == 0)
    def _():
        v_ref[...] = vc

    @pl.when(bh > 0)
    def _():
        v_ref[...] = v_ref[...] + vc

    @pl.when(bh == 0)
    def _():
        ent_ref[0, 0, 0] = ent

    @pl.when(bh > 0)
    def _():
        ent_ref[0, 0, 0] = ent_ref[0, 0, 0] + ent

    Lout = out_ref.shape[2]

    @pl.when(bh == 0)
    def _():
        for i in range(E):
            scr_ref[:, i * Lout:(i + 1) * Lout] = m_ref[:, i, :]

    gates = gates_t.T
    acc = gates[:, 0:1] * scr_ref[:, 0:Lout]
    for i in range(1, E):
        acc = acc + gates[:, i:i + 1] * scr_ref[:, i * Lout:(i + 1) * Lout]
    rows = lb * BL + jax.lax.broadcasted_iota(jnp.int32, acc.shape, 0)
    cols = jax.lax.broadcasted_iota(jnp.int32, acc.shape, 1)
    acc = jnp.where(rows == cols, acc + 1.0, acc)
    out_ref[0] = acc


def _loss_body(v_ref, ent_ref, loss_ref, *, n_rows_experts):
    v = v_ref[...]
    n = v.shape[0] * v.shape[1]
    mean = jnp.sum(v) / n
    var = jnp.sum((v - mean) ** 2) / (n - 1)
    loss_imp = var / (mean * mean + _CV_EPS)
    ent = ent_ref[0, 0, 0]
    for i in range(1, ent_ref.shape[0]):
        ent = ent + ent_ref[i, 0, 0]
    loss_dyn = ent / n_rows_experts
    loss_ref[0, 0] = loss_imp + 0.1 * loss_dyn


def kernel(x, masks, W_gate, W_noise):
    B, H, L, _ = x.shape
    E = W_gate.shape[0]
    BH = B * H
    BL = 512
    LB = L // BL
    xr = x.reshape(BH, L, L)

    out_bh, v, ent = pl.pallas_call(
        _main_body,
        grid=(LB, BH),
        in_specs=[
            pl.BlockSpec((1, BL, L), lambda lb, bh: (bh, lb, 0)),
            pl.BlockSpec((E, L), lambda lb, bh: (0, 0)),
            pl.BlockSpec((BL, E, L), lambda lb, bh: (lb, 0, 0)),
        ],
        out_specs=[
            pl.BlockSpec((1, BL, L), lambda lb, bh: (bh, lb, 0)),
            pl.BlockSpec((E, BL), lambda lb, bh: (0, lb)),
            pl.BlockSpec((1, 1, 1), lambda lb, bh: (lb, 0, 0),
                         memory_space=pltpu.SMEM),
        ],
        out_shape=[
            jax.ShapeDtypeStruct((BH, L, L), jnp.float32),
            jax.ShapeDtypeStruct((E, L), jnp.float32),
            jax.ShapeDtypeStruct((LB, 1, 1), jnp.float32),
        ],
        scratch_shapes=[pltpu.VMEM((BL, E * L), jnp.float32)],
        compiler_params=pltpu.CompilerParams(
            dimension_semantics=("parallel", "arbitrary"),
            vmem_limit_bytes=100 * 1024 * 1024),
    )(xr, W_gate, masks)

    loss2 = pl.pallas_call(
        functools.partial(_loss_body, n_rows_experts=BH * E),
        in_specs=[
            pl.BlockSpec((E, L), lambda: (0, 0)),
            pl.BlockSpec((LB, 1, 1), lambda: (0, 0, 0),
                         memory_space=pltpu.SMEM),
        ],
        out_specs=pl.BlockSpec((1, 1), lambda: (0, 0),
                               memory_space=pltpu.SMEM),
        out_shape=jax.ShapeDtypeStruct((1, 1), jnp.float32),
    )(v, ent)

    return out_bh.reshape(B, H, L, L), loss2[0, 0]


# speedup vs baseline: 4.9405x; 1.0155x over previous
import functools

import jax
import jax.numpy as jnp
from jax.experimental import pallas as pl
from jax.experimental.pallas import tpu as pltpu

TOP_P = 0.5
_LOG_EPS = 1e-10
_CV_EPS = 1e-10


def _main_body(x_ref, w_ref, m_ref, out_ref, v_ref, ent_ref, scr_ref):
    lb = pl.program_id(0)
    bhp = pl.program_id(1)
    w = w_ref[...]
    BL = x_ref.shape[1]
    E = w.shape[0]
    Lout = out_ref.shape[2]

    @pl.when(bhp == 0)
    def _():
        for i in range(E):
            scr_ref[:, i * Lout:(i + 1) * Lout] = m_ref[:, i, :]

    def gate_one(xb):
        lt = jax.lax.dot_general(
            w, xb, (((1,), (1,)), ((), ())),
            preferred_element_type=jnp.float32)
        mx = jnp.max(lt, axis=0, keepdims=True)
        ex = jnp.exp(lt - mx)
        p = ex / jnp.sum(ex, axis=0, keepdims=True)
        ent = -jnp.sum(p * jnp.log(p + _LOG_EPS))
        s_before_rows = []
        rank_rows = []
        for e in range(E):
            pe = p[e:e + 1, :]
            if e == 0:
                ahead = (p > pe)
            else:
                ahead = (p > pe) | ((p == pe) & (jax.lax.broadcasted_iota(
                    jnp.int32, p.shape, 0) < e))
            s_before_rows.append(
                jnp.sum(jnp.where(ahead, p, 0.0), axis=0, keepdims=True))
            rank_rows.append(
                jnp.sum(ahead.astype(jnp.float32), axis=0, keepdims=True))
        s_before = jnp.concatenate(s_before_rows, axis=0)
        rank = jnp.concatenate(rank_rows, axis=0)
        gates_t = (s_before <= TOP_P).astype(jnp.float32)
        pg = p * gates_t
        vc = jnp.concatenate(
            [jnp.sum(jnp.where(rank == r, pg, 0.0), axis=0, keepdims=True)
             for r in range(E)], axis=0)
        return gates_t, vc, ent

    gates_t0, vc0, ent0 = gate_one(x_ref[0])
    gates_t1, vc1, ent1 = gate_one(x_ref[1])
    vc = vc0 + vc1
    ent = ent0 + ent1

    @pl.when(bhp == 0)
    def _():
        v_ref[...] = vc

    @pl.when(bhp > 0)
    def _():
        v_ref[...] = v_ref[...] + vc

    @pl.when(bhp == 0)
    def _():
        ent_ref[0, 0, 0] = ent

    @pl.when(bhp > 0)
    def _():
        ent_ref[0, 0, 0] = ent_ref[0, 0, 0] + ent

    g0 = gates_t0.T
    g1 = gates_t1.T
    mb = scr_ref[:, 0:Lout]
    acc0 = g0[:, 0:1] * mb
    acc1 = g1[:, 0:1] * mb
    for i in range(1, E):
        mb = scr_ref[:, i * Lout:(i + 1) * Lout]
        acc0 = acc0 + g0[:, i:i + 1] * mb
        acc1 = acc1 + g1[:, i:i + 1] * mb
    rows = lb * BL + jax.lax.broadcasted_iota(jnp.int32, acc0.shape, 0)
    cols = jax.lax.broadcasted_iota(jnp.int32, acc0.shape, 1)
    eye = (rows == cols).astype(jnp.float32)
    out_ref[0] = acc0 + eye
    out_ref[1] = acc1 + eye


def _loss_body(v_ref, ent_ref, loss_ref, *, n_rows_experts):
    v = v_ref[...]
    n = v.shape[0] * v.shape[1]
    mean = jnp.sum(v) / n
    var = jnp.sum((v - mean) ** 2) / (n - 1)
    loss_imp = var / (mean * mean + _CV_EPS)
    ent = ent_ref[0, 0, 0]
    for i in range(1, ent_ref.shape[0]):
        ent = ent + ent_ref[i, 0, 0]
    loss_dyn = ent / n_rows_experts
    loss_ref[0, 0] = loss_imp + 0.1 * loss_dyn


def kernel(x, masks, W_gate, W_noise):
    B, H, L, _ = x.shape
    E = W_gate.shape[0]
    BH = B * H
    BL = 256
    LB = L // BL
    xr = x.reshape(BH, L, L)

    out_bh, v, ent = pl.pallas_call(
        _main_body,
        grid=(LB, BH // 2),
        in_specs=[
            pl.BlockSpec((2, BL, L), lambda lb, bh: (bh, lb, 0)),
            pl.BlockSpec((E, L), lambda lb, bh: (0, 0)),
            pl.BlockSpec((BL, E, L), lambda lb, bh: (lb, 0, 0)),
        ],
        out_specs=[
            pl.BlockSpec((2, BL, L), lambda lb, bh: (bh, lb, 0)),
            pl.BlockSpec((E, BL), lambda lb, bh: (0, lb)),
            pl.BlockSpec((1, 1, 1), lambda lb, bh: (lb, 0, 0),
                         memory_space=pltpu.SMEM),
        ],
        out_shape=[
            jax.ShapeDtypeStruct((BH, L, L), jnp.float32),
            jax.ShapeDtypeStruct((E, L), jnp.float32),
            jax.ShapeDtypeStruct((LB, 1, 1), jnp.float32),
        ],
        scratch_shapes=[pltpu.VMEM((BL, E * L), jnp.float32)],
        compiler_params=pltpu.CompilerParams(
            dimension_semantics=("parallel", "arbitrary"),
            vmem_limit_bytes=100 * 1024 * 1024),
    )(xr, W_gate, masks)

    loss2 = pl.pallas_call(
        functools.partial(_loss_body, n_rows_experts=BH * E),
        in_specs=[
            pl.BlockSpec((E, L), lambda: (0, 0)),
            pl.BlockSpec((LB, 1, 1), lambda: (0, 0, 0),
                         memory_space=pltpu.SMEM),
        ],
        out_specs=pl.BlockSpec((1, 1), lambda: (0, 0),
                               memory_space=pltpu.SMEM),
        out_shape=jax.ShapeDtypeStruct((1, 1), jnp.float32),
    )(v, ent)

    return out_bh.reshape(B, H, L, L), loss2[0, 0]


# speedup vs baseline: 5.5756x; 1.1286x over previous
import functools

import jax
import jax.numpy as jnp
from jax.experimental import pallas as pl
from jax.experimental.pallas import tpu as pltpu

TOP_P = 0.5
_LOG_EPS = 1e-10
_CV_EPS = 1e-10


def _main_body(x_ref, w_ref, m_ref, out_ref, v_ref, ent_ref, scr_ref):
    lb = pl.program_id(0)
    bhp = pl.program_id(1)
    w = w_ref[...]
    BL = x_ref.shape[1]
    E = w.shape[0]
    Lout = out_ref.shape[2]

    @pl.when(bhp == 0)
    def _():
        for i in range(E):
            scr_ref[:, i * Lout:(i + 1) * Lout] = m_ref[:, i, :]

    def gate_one(xb):
        lt = jax.lax.dot_general(
            w, xb, (((1,), (1,)), ((), ())),
            preferred_element_type=jnp.float32)
        mx = jnp.max(lt, axis=0, keepdims=True)
        ex = jnp.exp(lt - mx)
        p = ex / jnp.sum(ex, axis=0, keepdims=True)
        ent = -jnp.sum(p * jnp.log(p + _LOG_EPS))
        s_before_rows = []
        rank_rows = []
        for e in range(E):
            pe = p[e:e + 1, :]
            if e == 0:
                ahead = (p > pe)
            else:
                ahead = (p > pe) | ((p == pe) & (jax.lax.broadcasted_iota(
                    jnp.int32, p.shape, 0) < e))
            s_before_rows.append(
                jnp.sum(jnp.where(ahead, p, 0.0), axis=0, keepdims=True))
            rank_rows.append(
                jnp.sum(ahead.astype(jnp.float32), axis=0, keepdims=True))
        s_before = jnp.concatenate(s_before_rows, axis=0)
        rank = jnp.concatenate(rank_rows, axis=0)
        gates_t = (s_before <= TOP_P).astype(jnp.float32)
        pg = p * gates_t
        vc = jnp.concatenate(
            [jnp.sum(jnp.where(rank == r, pg, 0.0), axis=0, keepdims=True)
             for r in range(E)], axis=0)
        return gates_t, vc, ent

    NB = x_ref.shape[0]
    gvs = [gate_one(x_ref[s]) for s in range(NB)]
    vc = gvs[0][1]
    ent = gvs[0][2]
    for s in range(1, NB):
        vc = vc + gvs[s][1]
        ent = ent + gvs[s][2]

    @pl.when(bhp == 0)
    def _():
        v_ref[...] = vc

    @pl.when(bhp > 0)
    def _():
        v_ref[...] = v_ref[...] + vc

    @pl.when(bhp == 0)
    def _():
        ent_ref[0, 0, 0] = ent

    @pl.when(bhp > 0)
    def _():
        ent_ref[0, 0, 0] = ent_ref[0, 0, 0] + ent

    gs = [gv[0].T for gv in gvs]
    mb = scr_ref[:, 0:Lout]
    accs = [g[:, 0:1] * mb for g in gs]
    for i in range(1, E):
        mb = scr_ref[:, i * Lout:(i + 1) * Lout]
        for s in range(NB):
            accs[s] = accs[s] + gs[s][:, i:i + 1] * mb
    rows = lb * BL + jax.lax.broadcasted_iota(jnp.int32, accs[0].shape, 0)
    cols = jax.lax.broadcasted_iota(jnp.int32, accs[0].shape, 1)
    eye = (rows == cols).astype(jnp.float32)
    for s in range(NB):
        out_ref[s] = accs[s] + eye


def _loss_body(v_ref, ent_ref, loss_ref, *, n_rows_experts):
    v = v_ref[...]
    n = v.shape[0] * v.shape[1]
    mean = jnp.sum(v) / n
    var = jnp.sum((v - mean) ** 2) / (n - 1)
    loss_imp = var / (mean * mean + _CV_EPS)
    ent = ent_ref[0, 0, 0]
    for i in range(1, ent_ref.shape[0]):
        ent = ent + ent_ref[i, 0, 0]
    loss_dyn = ent / n_rows_experts
    loss_ref[0, 0] = loss_imp + 0.1 * loss_dyn


def kernel(x, masks, W_gate, W_noise):
    B, H, L, _ = x.shape
    E = W_gate.shape[0]
    BH = B * H
    BL = 256
    NBH = 4
    LB = L // BL
    xr = x.reshape(BH, L, L)

    out_bh, v, ent = pl.pallas_call(
        _main_body,
        grid=(LB, BH // NBH),
        in_specs=[
            pl.BlockSpec((NBH, BL, L), lambda lb, bh: (bh, lb, 0)),
            pl.BlockSpec((E, L), lambda lb, bh: (0, 0)),
            pl.BlockSpec((BL, E, L), lambda lb, bh: (lb, 0, 0)),
        ],
        out_specs=[
            pl.BlockSpec((NBH, BL, L), lambda lb, bh: (bh, lb, 0)),
            pl.BlockSpec((E, BL), lambda lb, bh: (0, lb)),
            pl.BlockSpec((1, 1, 1), lambda lb, bh: (lb, 0, 0),
                         memory_space=pltpu.SMEM),
        ],
        out_shape=[
            jax.ShapeDtypeStruct((BH, L, L), jnp.float32),
            jax.ShapeDtypeStruct((E, L), jnp.float32),
            jax.ShapeDtypeStruct((LB, 1, 1), jnp.float32),
        ],
        scratch_shapes=[pltpu.VMEM((BL, E * L), jnp.float32)],
        compiler_params=pltpu.CompilerParams(
            dimension_semantics=("parallel", "arbitrary"),
            vmem_limit_bytes=100 * 1024 * 1024),
    )(xr, W_gate, masks)

    loss2 = pl.pallas_call(
        functools.partial(_loss_body, n_rows_experts=BH * E),
        in_specs=[
            pl.BlockSpec((E, L), lambda: (0, 0)),
            pl.BlockSpec((LB, 1, 1), lambda: (0, 0, 0),
                         memory_space=pltpu.SMEM),
        ],
        out_specs=pl.BlockSpec((1, 1), lambda: (0, 0),
                               memory_space=pltpu.SMEM),
        out_shape=jax.ShapeDtypeStruct((1, 1), jnp.float32),
    )(v, ent)

    return out_bh.reshape(B, H, L, L), loss2[0, 0]


# speedup vs baseline: 5.9186x; 1.0615x over previous
import functools

import jax
import jax.numpy as jnp
from jax.experimental import pallas as pl
from jax.experimental.pallas import tpu as pltpu

TOP_P = 0.5
_LOG_EPS = 1e-10
_CV_EPS = 1e-10


def _main_body(x_ref, w_ref, m_hbm, out_ref, v_ref, ent_ref, scr_ref, sem):
    lb = pl.program_id(0)
    bhp = pl.program_id(1)
    LBn = pl.num_programs(0)
    w = w_ref[...]
    BL = x_ref.shape[1]
    E = w.shape[0]
    cur = jax.lax.rem(lb, 2)
    nxt = jax.lax.rem(lb + 1, 2)

    def masks_group_copy(group, slot, i):
        return pltpu.make_async_copy(
            m_hbm.at[pl.ds(group * BL, BL), i, :],
            scr_ref.at[slot, i],
            sem.at[slot])

    @pl.when(bhp == 0)
    def _():
        @pl.when(lb == 0)
        def _():
            for i in range(E):
                masks_group_copy(lb, cur, i).start()
        for i in range(E):
            masks_group_copy(lb, cur, i).wait()

        @pl.when(lb + 1 < LBn)
        def _():
            for i in range(E):
                masks_group_copy(lb + 1, nxt, i).start()

    def gate_one(xb):
        lt = jax.lax.dot_general(
            w, xb, (((1,), (1,)), ((), ())),
            preferred_element_type=jnp.float32)
        mx = jnp.max(lt, axis=0, keepdims=True)
        ex = jnp.exp(lt - mx)
        p = ex / jnp.sum(ex, axis=0, keepdims=True)
        ent = -jnp.sum(p * jnp.log(p + _LOG_EPS))
        s_before_rows = []
        rank_rows = []
        for e in range(E):
            pe = p[e:e + 1, :]
            if e == 0:
                ahead = (p > pe)
            else:
                ahead = (p > pe) | ((p == pe) & (jax.lax.broadcasted_iota(
                    jnp.int32, p.shape, 0) < e))
            s_before_rows.append(
                jnp.sum(jnp.where(ahead, p, 0.0), axis=0, keepdims=True))
            rank_rows.append(
                jnp.sum(ahead.astype(jnp.float32), axis=0, keepdims=True))
        s_before = jnp.concatenate(s_before_rows, axis=0)
        rank = jnp.concatenate(rank_rows, axis=0)
        gates_t = (s_before <= TOP_P).astype(jnp.float32)
        pg = p * gates_t
        vc = jnp.concatenate(
            [jnp.sum(jnp.where(rank == r, pg, 0.0), axis=0, keepdims=True)
             for r in range(E)], axis=0)
        return gates_t, vc, ent

    NB = x_ref.shape[0]
    gvs = [gate_one(x_ref[s]) for s in range(NB)]
    vc = gvs[0][1]
    ent = gvs[0][2]
    for s in range(1, NB):
        vc = vc + gvs[s][1]
        ent = ent + gvs[s][2]

    @pl.when(bhp == 0)
    def _():
        v_ref[...] = vc

    @pl.when(bhp > 0)
    def _():
        v_ref[...] = v_ref[...] + vc

    @pl.when(bhp == 0)
    def _():
        ent_ref[0, 0, 0] = ent

    @pl.when(bhp > 0)
    def _():
        ent_ref[0, 0, 0] = ent_ref[0, 0, 0] + ent

    gs = [gv[0].T for gv in gvs]
    mb = scr_ref[cur, 0]
    accs = [g[:, 0:1] * mb for g in gs]
    for i in range(1, E):
        mb = scr_ref[cur, i]
        for s in range(NB):
            accs[s] = accs[s] + gs[s][:, i:i + 1] * mb
    rows = lb * BL + jax.lax.broadcasted_iota(jnp.int32, accs[0].shape, 0)
    cols = jax.lax.broadcasted_iota(jnp.int32, accs[0].shape, 1)
    eye = (rows == cols).astype(jnp.float32)
    for s in range(NB):
        out_ref[s] = accs[s] + eye


def _loss_body(v_ref, ent_ref, loss_ref, *, n_rows_experts):
    v = v_ref[...]
    n = v.shape[0] * v.shape[1]
    mean = jnp.sum(v) / n
    var = jnp.sum((v - mean) ** 2) / (n - 1)
    loss_imp = var / (mean * mean + _CV_EPS)
    ent = ent_ref[0, 0, 0]
    for i in range(1, ent_ref.shape[0]):
        ent = ent + ent_ref[i, 0, 0]
    loss_dyn = ent / n_rows_experts
    loss_ref[0, 0] = loss_imp + 0.1 * loss_dyn


def kernel(x, masks, W_gate, W_noise):
    B, H, L, _ = x.shape
    E = W_gate.shape[0]
    BH = B * H
    BL = 256
    NBH = 4
    LB = L // BL
    xr = x.reshape(BH, L, L)

    out_bh, v, ent = pl.pallas_call(
        _main_body,
        grid=(LB, BH // NBH),
        in_specs=[
            pl.BlockSpec((NBH, BL, L), lambda lb, bh: (bh, lb, 0)),
            pl.BlockSpec((E, L), lambda lb, bh: (0, 0)),
            pl.BlockSpec(memory_space=pltpu.MemorySpace.HBM),
        ],
        out_specs=[
            pl.BlockSpec((NBH, BL, L), lambda lb, bh: (bh, lb, 0)),
            pl.BlockSpec((E, BL), lambda lb, bh: (0, lb)),
            pl.BlockSpec((1, 1, 1), lambda lb, bh: (lb, 0, 0),
                         memory_space=pltpu.SMEM),
        ],
        out_shape=[
            jax.ShapeDtypeStruct((BH, L, L), jnp.float32),
            jax.ShapeDtypeStruct((E, L), jnp.float32),
            jax.ShapeDtypeStruct((LB, 1, 1), jnp.float32),
        ],
        scratch_shapes=[
            pltpu.VMEM((2, E, BL, L), jnp.float32),
            pltpu.SemaphoreType.DMA((2,)),
        ],
        compiler_params=pltpu.CompilerParams(
            dimension_semantics=("arbitrary", "arbitrary"),
            vmem_limit_bytes=100 * 1024 * 1024),
    )(xr, W_gate, masks)

    loss2 = pl.pallas_call(
        functools.partial(_loss_body, n_rows_experts=BH * E),
        in_specs=[
            pl.BlockSpec((E, L), lambda: (0, 0)),
            pl.BlockSpec((LB, 1, 1), lambda: (0, 0, 0),
                         memory_space=pltpu.SMEM),
        ],
        out_specs=pl.BlockSpec((1, 1), lambda: (0, 0),
                               memory_space=pltpu.SMEM),
        out_shape=jax.ShapeDtypeStruct((1, 1), jnp.float32),
    )(v, ent)

    return out_bh.reshape(B, H, L, L), loss2[0, 0]


# speedup vs baseline: 5.9817x; 1.0107x over previous
import functools

import jax
import jax.numpy as jnp
from jax.experimental import pallas as pl
from jax.experimental.pallas import tpu as pltpu

TOP_P = 0.5
_LOG_EPS = 1e-10
_CV_EPS = 1e-10


def _main_body(x_ref, w_ref, m_hbm, out_ref, v_ref, ent_ref, scr_ref, sem):
    lb = pl.program_id(0)
    bhp = pl.program_id(1)
    LBn = pl.num_programs(0)
    w = w_ref[...]
    BL = x_ref.shape[1]
    E = w.shape[0]
    cur = jax.lax.rem(lb, 2)
    nxt = jax.lax.rem(lb + 1, 2)

    def masks_group_copy(group, slot, i):
        return pltpu.make_async_copy(
            m_hbm.at[pl.ds(group * BL, BL), i, :],
            scr_ref.at[slot, i],
            sem.at[slot])

    @pl.when(bhp == 0)
    def _():
        @pl.when(lb == 0)
        def _():
            for i in range(E):
                masks_group_copy(lb, cur, i).start()
        for i in range(E):
            masks_group_copy(lb, cur, i).wait()

        @pl.when(lb + 1 < LBn)
        def _():
            for i in range(E):
                masks_group_copy(lb + 1, nxt, i).start()

    def gate_one(xb):
        lt = jax.lax.dot_general(
            w, xb, (((1,), (1,)), ((), ())),
            preferred_element_type=jnp.float32)
        mx = jnp.max(lt, axis=0, keepdims=True)
        ex = jnp.exp(lt - mx)
        p = ex / jnp.sum(ex, axis=0, keepdims=True)
        ent = -jnp.sum(p * jnp.log(p + _LOG_EPS))
        s_before_rows = []
        rank_rows = []
        for e in range(E):
            pe = p[e:e + 1, :]
            if e == 0:
                ahead = (p > pe)
            else:
                ahead = (p > pe) | ((p == pe) & (jax.lax.broadcasted_iota(
                    jnp.int32, p.shape, 0) < e))
            s_before_rows.append(
                jnp.sum(jnp.where(ahead, p, 0.0), axis=0, keepdims=True))
            rank_rows.append(
                jnp.sum(ahead.astype(jnp.float32), axis=0, keepdims=True))
        s_before = jnp.concatenate(s_before_rows, axis=0)
        rank = jnp.concatenate(rank_rows, axis=0)
        gates_t = (s_before <= TOP_P).astype(jnp.float32)
        pg = p * gates_t
        vc = jnp.concatenate(
            [jnp.sum(jnp.where(rank == r, pg, 0.0), axis=0, keepdims=True)
             for r in range(E)], axis=0)
        return gates_t, vc, ent

    NB = x_ref.shape[0]
    gvs = [gate_one(x_ref[s]) for s in range(NB)]
    vc = gvs[0][1]
    ent = gvs[0][2]
    for s in range(1, NB):
        vc = vc + gvs[s][1]
        ent = ent + gvs[s][2]

    @pl.when(bhp == 0)
    def _():
        v_ref[...] = vc

    @pl.when(bhp > 0)
    def _():
        v_ref[...] = v_ref[...] + vc

    @pl.when(bhp == 0)
    def _():
        ent_ref[0, 0, 0] = ent

    @pl.when(bhp > 0)
    def _():
        ent_ref[0, 0, 0] = ent_ref[0, 0, 0] + ent

    gs = [gv[0].T for gv in gvs]
    mb = scr_ref[cur, 0]
    accs = [g[:, 0:1] * mb for g in gs]
    for i in range(1, E):
        mb = scr_ref[cur, i]
        for s in range(NB):
            accs[s] = accs[s] + gs[s][:, i:i + 1] * mb
    rows = lb * BL + jax.lax.broadcasted_iota(jnp.int32, accs[0].shape, 0)
    cols = jax.lax.broadcasted_iota(jnp.int32, accs[0].shape, 1)
    eye = (rows == cols).astype(jnp.float32)
    for s in range(NB):
        out_ref[s] = accs[s] + eye


def _loss_body(v_ref, ent_ref, loss_ref, *, n_rows_experts):
    v = v_ref[...]
    n = v.shape[0] * v.shape[1]
    mean = jnp.sum(v) / n
    var = jnp.sum((v - mean) ** 2) / (n - 1)
    loss_imp = var / (mean * mean + _CV_EPS)
    ent = ent_ref[0, 0, 0]
    for i in range(1, ent_ref.shape[0]):
        ent = ent + ent_ref[i, 0, 0]
    loss_dyn = ent / n_rows_experts
    loss_ref[0, 0] = loss_imp + 0.1 * loss_dyn


def kernel(x, masks, W_gate, W_noise):
    B, H, L, _ = x.shape
    E = W_gate.shape[0]
    BH = B * H
    BL = 256
    NBH = 8
    LB = L // BL
    xr = x.reshape(BH, L, L)

    out_bh, v, ent = pl.pallas_call(
        _main_body,
        grid=(LB, BH // NBH),
        in_specs=[
            pl.BlockSpec((NBH, BL, L), lambda lb, bh: (bh, lb, 0)),
            pl.BlockSpec((E, L), lambda lb, bh: (0, 0)),
            pl.BlockSpec(memory_space=pltpu.MemorySpace.HBM),
        ],
        out_specs=[
            pl.BlockSpec((NBH, BL, L), lambda lb, bh: (bh, lb, 0)),
            pl.BlockSpec((E, BL), lambda lb, bh: (0, lb)),
            pl.BlockSpec((1, 1, 1), lambda lb, bh: (lb, 0, 0),
                         memory_space=pltpu.SMEM),
        ],
        out_shape=[
            jax.ShapeDtypeStruct((BH, L, L), jnp.float32),
            jax.ShapeDtypeStruct((E, L), jnp.float32),
            jax.ShapeDtypeStruct((LB, 1, 1), jnp.float32),
        ],
        scratch_shapes=[
            pltpu.VMEM((2, E, BL, L), jnp.float32),
            pltpu.SemaphoreType.DMA((2,)),
        ],
        compiler_params=pltpu.CompilerParams(
            dimension_semantics=("arbitrary", "arbitrary"),
            vmem_limit_bytes=100 * 1024 * 1024),
    )(xr, W_gate, masks)

    loss2 = pl.pallas_call(
        functools.partial(_loss_body, n_rows_experts=BH * E),
        in_specs=[
            pl.BlockSpec((E, L), lambda: (0, 0)),
            pl.BlockSpec((LB, 1, 1), lambda: (0, 0, 0),
                         memory_space=pltpu.SMEM),
        ],
        out_specs=pl.BlockSpec((1, 1), lambda: (0, 0),
                               memory_space=pltpu.SMEM),
        out_shape=jax.ShapeDtypeStruct((1, 1), jnp.float32),
    )(v, ent)

    return out_bh.reshape(B, H, L, L), loss2[0, 0]


# speedup vs baseline: 6.0484x; 1.0112x over previous
import functools

import jax
import jax.numpy as jnp
from jax.experimental import pallas as pl
from jax.experimental.pallas import tpu as pltpu

TOP_P = 0.5
_LOG_EPS = 1e-10
_CV_EPS = 1e-10


def _main_body(x_ref, w_ref, m_hbm, out_ref, v_ref, ent_ref, scr_ref, sem):
    lb = pl.program_id(0)
    bhp = pl.program_id(1)
    LBn = pl.num_programs(0)
    w = w_ref[...]
    BL = x_ref.shape[1]
    E = w.shape[0]
    cur = jax.lax.rem(lb, 2)
    nxt = jax.lax.rem(lb + 1, 2)

    def masks_group_copy(group, slot, i):
        return pltpu.make_async_copy(
            m_hbm.at[pl.ds(group * BL, BL), i, :],
            scr_ref.at[slot, i],
            sem.at[slot])

    @pl.when(bhp == 0)
    def _():
        @pl.when(lb == 0)
        def _():
            for i in range(E):
                masks_group_copy(lb, cur, i).start()
        for i in range(E):
            masks_group_copy(lb, cur, i).wait()

        @pl.when(lb + 1 < LBn)
        def _():
            for i in range(E):
                masks_group_copy(lb + 1, nxt, i).start()

    def gate_one(xb):
        lt = jax.lax.dot_general(
            w, xb, (((1,), (1,)), ((), ())),
            preferred_element_type=jnp.float32)
        mx = jnp.max(lt, axis=0, keepdims=True)
        ex = jnp.exp(lt - mx)
        p = ex / jnp.sum(ex, axis=0, keepdims=True)
        ent = -jnp.sum(p * jnp.log(p + _LOG_EPS))
        s_before_rows = []
        rank_rows = []
        for e in range(E):
            pe = p[e:e + 1, :]
            if e == 0:
                ahead = (p > pe)
            else:
                ahead = (p > pe) | ((p == pe) & (jax.lax.broadcasted_iota(
                    jnp.int32, p.shape, 0) < e))
            s_before_rows.append(
                jnp.sum(jnp.where(ahead, p, 0.0), axis=0, keepdims=True))
            rank_rows.append(
                jnp.sum(ahead.astype(jnp.float32), axis=0, keepdims=True))
        s_before = jnp.concatenate(s_before_rows, axis=0)
        rank = jnp.concatenate(rank_rows, axis=0)
        gates_t = (s_before <= TOP_P).astype(jnp.float32)
        pg = p * gates_t
        vc = jnp.concatenate(
            [jnp.sum(jnp.where(rank == r, pg, 0.0), axis=0, keepdims=True)
             for r in range(E)], axis=0)
        return gates_t, vc, ent

    NB = x_ref.shape[0]
    gvs = [gate_one(x_ref[s]) for s in range(NB)]
    vc = gvs[0][1]
    ent = gvs[0][2]
    for s in range(1, NB):
        vc = vc + gvs[s][1]
        ent = ent + gvs[s][2]

    @pl.when(bhp == 0)
    def _():
        v_ref[...] = vc

    @pl.when(bhp > 0)
    def _():
        v_ref[...] = v_ref[...] + vc

    @pl.when(bhp == 0)
    def _():
        ent_ref[0, 0, 0] = ent

    @pl.when(bhp > 0)
    def _():
        ent_ref[0, 0, 0] = ent_ref[0, 0, 0] + ent

    gs = [gv[0].T for gv in gvs]
    mb = scr_ref[cur, 0]
    accs = [g[:, 0:1] * mb for g in gs]
    for i in range(1, E):
        mb = scr_ref[cur, i]
        for s in range(NB):
            accs[s] = accs[s] + gs[s][:, i:i + 1] * mb
    rows = lb * BL + jax.lax.broadcasted_iota(jnp.int32, accs[0].shape, 0)
    cols = jax.lax.broadcasted_iota(jnp.int32, accs[0].shape, 1)
    eye = (rows == cols).astype(jnp.float32)
    for s in range(NB):
        out_ref[s] = accs[s] + eye


def _loss_body(v_ref, ent_ref, loss_ref, *, n_rows_experts):
    v = v_ref[...]
    n = v.shape[0] * v.shape[1]
    mean = jnp.sum(v) / n
    var = jnp.sum((v - mean) ** 2) / (n - 1)
    loss_imp = var / (mean * mean + _CV_EPS)
    ent = ent_ref[0, 0, 0]
    for i in range(1, ent_ref.shape[0]):
        ent = ent + ent_ref[i, 0, 0]
    loss_dyn = ent / n_rows_experts
    loss_ref[0, 0] = loss_imp + 0.1 * loss_dyn


def kernel(x, masks, W_gate, W_noise):
    B, H, L, _ = x.shape
    E = W_gate.shape[0]
    BH = B * H
    BL = 128
    NBH = 8
    LB = L // BL
    xr = x.reshape(BH, L, L)

    out_bh, v, ent = pl.pallas_call(
        _main_body,
        grid=(LB, BH // NBH),
        in_specs=[
            pl.BlockSpec((NBH, BL, L), lambda lb, bh: (bh, lb, 0)),
            pl.BlockSpec((E, L), lambda lb, bh: (0, 0)),
            pl.BlockSpec(memory_space=pltpu.MemorySpace.HBM),
        ],
        out_specs=[
            pl.BlockSpec((NBH, BL, L), lambda lb, bh: (bh, lb, 0)),
            pl.BlockSpec((E, BL), lambda lb, bh: (0, lb)),
            pl.BlockSpec((1, 1, 1), lambda lb, bh: (lb, 0, 0),
                         memory_space=pltpu.SMEM),
        ],
        out_shape=[
            jax.ShapeDtypeStruct((BH, L, L), jnp.float32),
            jax.ShapeDtypeStruct((E, L), jnp.float32),
            jax.ShapeDtypeStruct((LB, 1, 1), jnp.float32),
        ],
        scratch_shapes=[
            pltpu.VMEM((2, E, BL, L), jnp.float32),
            pltpu.SemaphoreType.DMA((2,)),
        ],
        compiler_params=pltpu.CompilerParams(
            dimension_semantics=("arbitrary", "arbitrary"),
            vmem_limit_bytes=100 * 1024 * 1024),
    )(xr, W_gate, masks)

    loss2 = pl.pallas_call(
        functools.partial(_loss_body, n_rows_experts=BH * E),
        in_specs=[
            pl.BlockSpec((E, L), lambda: (0, 0)),
            pl.BlockSpec((LB, 1, 1), lambda: (0, 0, 0),
                         memory_space=pltpu.SMEM),
        ],
        out_specs=pl.BlockSpec((1, 1), lambda: (0, 0),
                               memory_space=pltpu.SMEM),
        out_shape=jax.ShapeDtypeStruct((1, 1), jnp.float32),
    )(v, ent)

    return out_bh.reshape(B, H, L, L), loss2[0, 0]


# speedup vs baseline: 6.2118x; 1.0270x over previous
import functools

import jax
import jax.numpy as jnp
from jax.experimental import pallas as pl
from jax.experimental.pallas import tpu as pltpu

TOP_P = 0.5
_LOG_EPS = 1e-10
_CV_EPS = 1e-10


def _main_body(x_ref, w_ref, m_hbm, out_ref, v_ref, ent_ref, scr_ref, sem):
    lb = pl.program_id(0)
    bhp = pl.program_id(1)
    LBn = pl.num_programs(0)
    w = w_ref[...]
    BL = x_ref.shape[1]
    E = w.shape[0]
    cur = jax.lax.rem(lb, 2)
    nxt = jax.lax.rem(lb + 1, 2)

    def masks_group_copy(group, slot, i):
        return pltpu.make_async_copy(
            m_hbm.at[pl.ds(group * BL, BL), i, :],
            scr_ref.at[slot, i],
            sem.at[slot])

    @pl.when(bhp == 0)
    def _():
        @pl.when(lb == 0)
        def _():
            for i in range(E):
                masks_group_copy(lb, cur, i).start()
        for i in range(E):
            masks_group_copy(lb, cur, i).wait()

        @pl.when(lb + 1 < LBn)
        def _():
            for i in range(E):
                masks_group_copy(lb + 1, nxt, i).start()

    def gate_one(xb):
        lt = jax.lax.dot_general(
            w, xb, (((1,), (1,)), ((), ())),
            preferred_element_type=jnp.float32)
        mx = jnp.max(lt, axis=0, keepdims=True)
        ex = jnp.exp(lt - mx)
        p = ex / jnp.sum(ex, axis=0, keepdims=True)
        ent = -jnp.sum(p * jnp.log(p + _LOG_EPS))
        s_before_rows = []
        rank_rows = []
        for e in range(E):
            pe = p[e:e + 1, :]
            if e == 0:
                ahead = (p > pe)
            else:
                ahead = (p > pe) | ((p == pe) & (jax.lax.broadcasted_iota(
                    jnp.int32, p.shape, 0) < e))
            s_before_rows.append(
                jnp.sum(jnp.where(ahead, p, 0.0), axis=0, keepdims=True))
            rank_rows.append(
                jnp.sum(ahead.astype(jnp.float32), axis=0, keepdims=True))
        s_before = jnp.concatenate(s_before_rows, axis=0)
        rank = jnp.concatenate(rank_rows, axis=0)
        gates_t = (s_before <= TOP_P).astype(jnp.float32)
        pg = p * gates_t
        vc = jnp.concatenate(
            [jnp.sum(jnp.where(rank == r, pg, 0.0), axis=0, keepdims=True)
             for r in range(E)], axis=0)
        return gates_t, vc, ent

    NB = x_ref.shape[0]
    gvs = [gate_one(x_ref[s]) for s in range(NB)]
    vc = gvs[0][1]
    ent = gvs[0][2]
    for s in range(1, NB):
        vc = vc + gvs[s][1]
        ent = ent + gvs[s][2]

    @pl.when(bhp == 0)
    def _():
        v_ref[...] = vc

    @pl.when(bhp > 0)
    def _():
        v_ref[...] = v_ref[...] + vc

    @pl.when(bhp == 0)
    def _():
        ent_ref[0, 0, 0] = ent

    @pl.when(bhp > 0)
    def _():
        ent_ref[0, 0, 0] = ent_ref[0, 0, 0] + ent

    gs = [gv[0].T for gv in gvs]
    mb = scr_ref[cur, 0]
    accs = [g[:, 0:1] * mb for g in gs]
    for i in range(1, E):
        mb = scr_ref[cur, i]
        for s in range(NB):
            accs[s] = accs[s] + gs[s][:, i:i + 1] * mb
    rows = lb * BL + jax.lax.broadcasted_iota(jnp.int32, accs[0].shape, 0)
    cols = jax.lax.broadcasted_iota(jnp.int32, accs[0].shape, 1)
    eye = (rows == cols).astype(jnp.float32)
    for s in range(NB):
        out_ref[s] = accs[s] + eye


def _loss_body(v_ref, ent_ref, loss_ref, *, n_rows_experts):
    v = v_ref[...]
    n = v.shape[0] * v.shape[1]
    mean = jnp.sum(v) / n
    var = jnp.sum((v - mean) ** 2) / (n - 1)
    loss_imp = var / (mean * mean + _CV_EPS)
    ent = ent_ref[0, 0, 0]
    for i in range(1, ent_ref.shape[0]):
        ent = ent + ent_ref[i, 0, 0]
    loss_dyn = ent / n_rows_experts
    loss_ref[0, 0] = loss_imp + 0.1 * loss_dyn


def kernel(x, masks, W_gate, W_noise):
    B, H, L, _ = x.shape
    E = W_gate.shape[0]
    BH = B * H
    BL = 128
    NBH = 16
    LB = L // BL
    xr = x.reshape(BH, L, L)

    out_bh, v, ent = pl.pallas_call(
        _main_body,
        grid=(LB, BH // NBH),
        in_specs=[
            pl.BlockSpec((NBH, BL, L), lambda lb, bh: (bh, lb, 0)),
            pl.BlockSpec((E, L), lambda lb, bh: (0, 0)),
            pl.BlockSpec(memory_space=pltpu.MemorySpace.HBM),
        ],
        out_specs=[
            pl.BlockSpec((NBH, BL, L), lambda lb, bh: (bh, lb, 0)),
            pl.BlockSpec((E, BL), lambda lb, bh: (0, lb)),
            pl.BlockSpec((1, 1, 1), lambda lb, bh: (lb, 0, 0),
                         memory_space=pltpu.SMEM),
        ],
        out_shape=[
            jax.ShapeDtypeStruct((BH, L, L), jnp.float32),
            jax.ShapeDtypeStruct((E, L), jnp.float32),
            jax.ShapeDtypeStruct((LB, 1, 1), jnp.float32),
        ],
        scratch_shapes=[
            pltpu.VMEM((2, E, BL, L), jnp.float32),
            pltpu.SemaphoreType.DMA((2,)),
        ],
        compiler_params=pltpu.CompilerParams(
            dimension_semantics=("arbitrary", "arbitrary"),
            vmem_limit_bytes=100 * 1024 * 1024),
    )(xr, W_gate, masks)

    loss2 = pl.pallas_call(
        functools.partial(_loss_body, n_rows_experts=BH * E),
        in_specs=[
            pl.BlockSpec((E, L), lambda: (0, 0)),
            pl.BlockSpec((LB, 1, 1), lambda: (0, 0, 0),
                         memory_space=pltpu.SMEM),
        ],
        out_specs=pl.BlockSpec((1, 1), lambda: (0, 0),
                               memory_space=pltpu.SMEM),
        out_shape=jax.ShapeDtypeStruct((1, 1), jnp.float32),
    )(v, ent)

    return out_bh.reshape(B, H, L, L), loss2[0, 0]
